# Initial kernel scaffold; baseline (speedup 1.0000x reference)
import jax
import jax.numpy as jnp
from jax.experimental import pallas as pl


def kernel(strands, strand_embed):
    raise NotImplementedError("write your pallas kernel here")


# speedup vs baseline: 1.7302x; 1.7302x over previous
import functools

import jax
import jax.numpy as jnp
from jax import lax
from jax.experimental import pallas as pl
from jax.experimental.pallas import tpu as pltpu
from jax.experimental.pallas import tpu_sc as plsc

D_MODEL = 64
BATCH = 4096
N_MOTIFS = 200

NC, NS = 2, 16
NW = NC * NS
TOTAL = BATCH * N_MOTIFS
QUAD = 4
ROW_Q = 128
NQ = TOTAL // QUAD
ROWS = NQ // ROW_Q
ROWS_PER_W = ROWS // NW
TAB_R = 2 ** QUAD
TAB_C = QUAD * D_MODEL
FUSE_BLK = 160


def _build_table_body(e_ref, t_ref):
    e = e_ref[...]
    big = jnp.concatenate([e, e, e, e], axis=1)
    b0 = jnp.broadcast_to(big[0:1, :], (TAB_R, TAB_C))
    b1 = jnp.broadcast_to(big[1:2, :], (TAB_R, TAB_C))
    r = lax.broadcasted_iota(jnp.int32, (TAB_R, TAB_C), 0)
    c = lax.broadcasted_iota(jnp.int32, (TAB_R, TAB_C), 1)
    bit = lax.shift_right_logical(r, (QUAD - 1) - c // D_MODEL) & 1
    t_ref[...] = jnp.where(bit == 1, b1, b0)


_build_table = pl.pallas_call(
    _build_table_body,
    out_shape=jax.ShapeDtypeStruct((TAB_R, TAB_C), jnp.float32),
)


def _fuse_idx_body(s_ref, q_ref):
    x = s_ref[...]
    q_ref[...] = ((x[0] * 2 + x[1]) * 2 + x[2]) * 2 + x[3]


_fuse_idx = pl.pallas_call(
    _fuse_idx_body,
    grid=(ROWS // FUSE_BLK,),
    in_specs=[pl.BlockSpec((QUAD, FUSE_BLK, ROW_Q), lambda i: (0, i, 0))],
    out_specs=pl.BlockSpec((FUSE_BLK, ROW_Q), lambda i: (i, 0)),
    out_shape=jax.ShapeDtypeStruct((ROWS, ROW_Q), jnp.int32),
)


@functools.partial(
    pl.kernel,
    out_type=jax.ShapeDtypeStruct((ROWS, ROW_Q, TAB_C), jnp.float32),
    mesh=plsc.VectorSubcoreMesh(
        core_axis_name="c", subcore_axis_name="s",
        num_cores=NC, num_subcores=NS),
    scratch_types=[
        pltpu.VMEM((ROWS_PER_W, ROW_Q), jnp.int32),
        pltpu.VMEM((2, ROW_Q, TAB_C), jnp.float32),
        pltpu.SemaphoreType.DMA,
        pltpu.SemaphoreType.DMA,
    ],
)
def _embed_lookup(q_hbm, tab_hbm, out_hbm, idx_v, rows_v, gsem, osem):
    wid = lax.axis_index("s") * NC + lax.axis_index("c")
    base0 = wid * ROWS_PER_W

    pltpu.sync_copy(q_hbm.at[wid], idx_v)
    pltpu.async_copy(tab_hbm.at[idx_v.at[0]], rows_v.at[0], gsem)

    def step(i, carry):
        buf = lax.rem(i, 2)
        nbuf = 1 - buf
        pltpu.make_async_copy(
            tab_hbm.at[idx_v.at[i]], rows_v.at[buf], gsem).wait()

        @pl.when(i >= 1)
        def _():
            pltpu.make_async_copy(
                rows_v.at[nbuf], out_hbm.at[base0 + i - 1], osem).wait()

        @pl.when(i + 1 < ROWS_PER_W)
        def _():
            pltpu.async_copy(
                tab_hbm.at[idx_v.at[i + 1]], rows_v.at[nbuf], gsem)

        pltpu.async_copy(rows_v.at[buf], out_hbm.at[base0 + i], osem)
        return carry

    lax.fori_loop(0, ROWS_PER_W, step, 0)
    last = ROWS_PER_W - 1
    pltpu.make_async_copy(
        rows_v.at[lax.rem(last, 2)], out_hbm.at[base0 + last], osem).wait()


def kernel(strands, strand_embed):
    tab = _build_table(strand_embed)
    s4 = jnp.transpose(
        strands.astype(jnp.int32).reshape(NQ, QUAD)).reshape(
            QUAD, ROWS, ROW_Q)
    quads = _fuse_idx(s4).reshape(NW, ROWS_PER_W, ROW_Q)
    out = _embed_lookup(quads, tab)
    return out.reshape(BATCH, N_MOTIFS, D_MODEL)


# speedup vs baseline: 3.3978x; 1.9638x over previous
import functools

import jax
import jax.numpy as jnp
from jax import lax
from jax.experimental import pallas as pl
from jax.experimental.pallas import tpu as pltpu
from jax.experimental.pallas import tpu_sc as plsc

D_MODEL = 64
BATCH = 4096
N_MOTIFS = 200

NC, NS = 2, 16
NW = NC * NS
TOTAL = BATCH * N_MOTIFS
OCT = 8
ROW_I = 64
NI = TOTAL // OCT
ROWS = NI // ROW_I
ROWS_PER_W = ROWS // NW
NBUF = 3
TAB_R = 2 ** OCT
TAB_C = OCT * D_MODEL


def _build_table_body(e_ref, t_ref):
    e = e_ref[...]
    big = jnp.concatenate([e] * OCT, axis=1)
    b0 = jnp.broadcast_to(big[0:1, :], (TAB_R, TAB_C))
    b1 = jnp.broadcast_to(big[1:2, :], (TAB_R, TAB_C))
    r = lax.broadcasted_iota(jnp.int32, (TAB_R, TAB_C), 0)
    c = lax.broadcasted_iota(jnp.int32, (TAB_R, TAB_C), 1)
    bit = lax.shift_right_logical(r, (OCT - 1) - c // D_MODEL) & 1
    t_ref[...] = jnp.where(bit == 1, b1, b0)


_build_table = pl.pallas_call(
    _build_table_body,
    out_shape=jax.ShapeDtypeStruct((TAB_R, TAB_C), jnp.float32),
)


def _pair_pack_body(w_ref, p_ref):
    w = w_ref[...]
    p_ref[...] = (2 * (w & 1) + ((w >> 16) & 1)).astype(jnp.int8)


_pair_pack = pl.pallas_call(
    _pair_pack_body,
    out_shape=jax.ShapeDtypeStruct((TOTAL // 2 // 256, 256), jnp.int8),
)


def _oct_pack_body(w_ref, q_ref):
    w = w_ref[...]
    p0 = w & 3
    p1 = (w >> 8) & 3
    p2 = (w >> 16) & 3
    p3 = (w >> 24) & 3
    q_ref[...] = ((p0 * 4 + p1) * 4 + p2) * 4 + p3


_oct_pack = pl.pallas_call(
    _oct_pack_body,
    out_shape=jax.ShapeDtypeStruct((ROWS, ROW_I), jnp.int32),
)


@functools.partial(
    pl.kernel,
    out_type=jax.ShapeDtypeStruct((ROWS, ROW_I, TAB_C), jnp.float32),
    mesh=plsc.VectorSubcoreMesh(
        core_axis_name="c", subcore_axis_name="s",
        num_cores=NC, num_subcores=NS),
    scratch_types=[
        pltpu.VMEM((ROWS_PER_W, ROW_I), jnp.int32),
        pltpu.VMEM((NBUF, ROW_I, TAB_C), jnp.float32),
        pltpu.SemaphoreType.DMA,
        pltpu.SemaphoreType.DMA,
        pltpu.SemaphoreType.DMA,
    ],
)
def _embed_lookup(q_hbm, tab_hbm, out_hbm, idx_v, rows_v, gsem0, gsem1,
                  osem):
    wid = lax.axis_index("s") * NC + lax.axis_index("c")
    base0 = wid * ROWS_PER_W
    gsems = (gsem0, gsem1)

    pltpu.sync_copy(q_hbm.at[wid], idx_v)
    for j in range(NBUF - 1):
        pltpu.async_copy(tab_hbm.at[idx_v.at[j]], rows_v.at[j], gsems[j % 2])

    def step(i, carry):
        buf = lax.rem(i, NBUF)
        par = lax.rem(i, 2)
        nxt = i + NBUF - 1

        for p in range(2):
            @pl.when(par == p)
            def _(p=p):
                pltpu.make_async_copy(
                    tab_hbm.at[idx_v.at[i]], rows_v.at[buf],
                    gsems[p]).wait()

        @pl.when(i >= 1)
        def _():
            pltpu.make_async_copy(
                rows_v.at[buf], out_hbm.at[base0 + i - 1], osem).wait()

        for p in range(2):
            @pl.when((nxt < ROWS_PER_W) & (par == p))
            def _(p=p):
                pltpu.async_copy(
                    tab_hbm.at[idx_v.at[nxt]],
                    rows_v.at[lax.rem(nxt, NBUF)], gsems[p])

        pltpu.async_copy(rows_v.at[buf], out_hbm.at[base0 + i], osem)
        return carry

    lax.fori_loop(0, ROWS_PER_W, step, 0)
    last = ROWS_PER_W - 1
    pltpu.make_async_copy(
        rows_v.at[lax.rem(last, NBUF)], out_hbm.at[base0 + last], osem).wait()


def kernel(strands, strand_embed):
    tab = _build_table(strand_embed)
    w2 = lax.bitcast_convert_type(
        strands.astype(jnp.int16).reshape(TOTAL // 2 // 256, 256, 2),
        jnp.int32)
    pairs = _pair_pack(w2)
    w8 = lax.bitcast_convert_type(
        pairs.reshape(ROWS, ROW_I, 4), jnp.int32)
    octs = _oct_pack(w8).reshape(NW, ROWS_PER_W, ROW_I)
    out = _embed_lookup(octs, tab)
    return out.reshape(BATCH, N_MOTIFS, D_MODEL)


# speedup vs baseline: 4.8063x; 1.4145x over previous
import functools

import jax
import jax.numpy as jnp
from jax import lax
from jax.experimental import pallas as pl
from jax.experimental.pallas import tpu as pltpu
from jax.experimental.pallas import tpu_sc as plsc

D_MODEL = 64
BATCH = 4096
N_MOTIFS = 200

NC, NS = 2, 16
NW = NC * NS
B_PER_W = BATCH // NW
LANES = 16
MC = 4
N_CHUNK = N_MOTIFS // MC


@functools.partial(
    pl.kernel,
    out_type=jax.ShapeDtypeStruct((N_MOTIFS, D_MODEL, BATCH), jnp.float32),
    mesh=plsc.VectorSubcoreMesh(
        core_axis_name="c", subcore_axis_name="s",
        num_cores=NC, num_subcores=NS),
    scratch_types=[
        pltpu.VMEM((N_MOTIFS, B_PER_W), jnp.int32),
        pltpu.VMEM((2, D_MODEL, LANES), jnp.float32),
        pltpu.VMEM((MC, B_PER_W), jnp.float32),
        pltpu.VMEM((2, MC, D_MODEL, B_PER_W), jnp.float32),
        pltpu.SemaphoreType.DMA,
        pltpu.SemaphoreType.DMA,
    ],
)
def _strand_encode(s_hbm, tab_hbm, out_hbm, s_v, t_v, sf_v, out_v,
                   osem0, osem1):
    wid = lax.axis_index("s") * NC + lax.axis_index("c")
    b0 = wid * B_PER_W
    osems = (osem0, osem1)

    pltpu.sync_copy(s_hbm.at[:, pl.ds(b0, B_PER_W)], s_v)
    pltpu.sync_copy(tab_hbm, t_v)

    def chunk(i, carry):
        buf = lax.rem(i, 2)
        m0 = i * MC

        for p in range(2):
            @pl.when((i >= 2) & (lax.rem(i, 2) == p))
            def _(p=p):
                pltpu.make_async_copy(
                    out_v.at[buf],
                    out_hbm.at[pl.ds(m0, MC), :, pl.ds(b0, B_PER_W)],
                    osems[p]).wait()

        for mm in range(MC):
            for j in range(B_PER_W // LANES):
                sl = pl.ds(j * LANES, LANES)
                sf_v[mm, sl] = s_v[m0 + mm, sl].astype(jnp.float32)

        def col(d, c2):
            e0 = t_v[0, d, :]
            dl = t_v[1, d, :] - e0
            for mm in range(MC):
                for j in range(B_PER_W // LANES):
                    sl = pl.ds(j * LANES, LANES)
                    out_v[buf, mm, d, sl] = e0 + sf_v[mm, sl] * dl
            return c2

        lax.fori_loop(0, D_MODEL, col, 0)

        for p in range(2):
            @pl.when(lax.rem(i, 2) == p)
            def _(p=p):
                pltpu.async_copy(
                    out_v.at[buf],
                    out_hbm.at[pl.ds(m0, MC), :, pl.ds(b0, B_PER_W)],
                    osems[p])
        return carry

    lax.fori_loop(0, N_CHUNK, chunk, 0)
    for i in (N_CHUNK - 2, N_CHUNK - 1):
        pltpu.make_async_copy(
            out_v.at[lax.rem(i, 2)],
            out_hbm.at[pl.ds(i * MC, MC), :, pl.ds(b0, B_PER_W)],
            osems[i % 2]).wait()


def kernel(strands, strand_embed):
    s_t = strands.astype(jnp.int32).T
    tab = jnp.broadcast_to(
        strand_embed[:, :, None], (2, D_MODEL, LANES))
    out_t = _strand_encode(s_t, tab)
    return jnp.transpose(out_t, (2, 0, 1))


# speedup vs baseline: 24.2031x; 5.0357x over previous
import functools

import jax
import jax.numpy as jnp
from jax import lax
from jax.experimental import pallas as pl
from jax.experimental.pallas import tpu as pltpu
from jax.experimental.pallas import tpu_sc as plsc

D_MODEL = 64
BATCH = 4096
N_MOTIFS = 200

NC, NS = 2, 16
NW = NC * NS
B_PER_W = BATCH // NW
LANES = 16
MC = 4
N_CHUNK = N_MOTIFS // MC


@functools.partial(
    pl.kernel,
    out_type=jax.ShapeDtypeStruct((N_MOTIFS, D_MODEL, BATCH), jnp.float32),
    mesh=plsc.VectorSubcoreMesh(
        core_axis_name="c", subcore_axis_name="s",
        num_cores=NC, num_subcores=NS),
    scratch_types=[
        pltpu.VMEM((N_MOTIFS, B_PER_W), jnp.int32),
        pltpu.VMEM((2, D_MODEL, LANES), jnp.float32),
        pltpu.VMEM((MC, B_PER_W), jnp.float32),
        pltpu.VMEM((2, MC, D_MODEL, B_PER_W), jnp.float32),
        pltpu.SemaphoreType.DMA,
        pltpu.SemaphoreType.DMA,
    ],
)
def _strand_encode(s_hbm, tab_hbm, out_hbm, s_v, t_v, sf_v, out_v,
                   osem0, osem1):
    wid = lax.axis_index("s") * NC + lax.axis_index("c")
    b0 = wid * B_PER_W
    osems = (osem0, osem1)

    pltpu.sync_copy(s_hbm.at[:, pl.ds(b0, B_PER_W)], s_v)
    pltpu.sync_copy(tab_hbm, t_v)

    def out_slab(i):
        return out_hbm.at[pl.ds(i * MC, MC), :, pl.ds(b0, B_PER_W)]

    def half(ii, buf):
        i = 2 * ii + buf
        m0 = i * MC

        @pl.when(ii >= 1)
        def _():
            pltpu.make_async_copy(
                out_v.at[buf], out_slab(i - 2), osems[buf]).wait()

        for mm in range(MC):
            for j in range(B_PER_W // LANES):
                sl = pl.ds(j * LANES, LANES)
                sf_v[mm, sl] = s_v[m0 + mm, sl].astype(jnp.float32)

        @plsc.parallel_loop(0, D_MODEL, step=1, unroll=4)
        def _(d):
            e0 = t_v[0, d, :]
            dl = t_v[1, d, :] - e0
            for mm in range(MC):
                for j in range(B_PER_W // LANES):
                    sl = pl.ds(j * LANES, LANES)
                    out_v[buf, mm, d, sl] = e0 + sf_v[mm, sl] * dl

        pltpu.async_copy(out_v.at[buf], out_slab(i), osems[buf])

    def chunk_pair(ii, carry):
        half(ii, 0)
        half(ii, 1)
        return carry

    lax.fori_loop(0, N_CHUNK // 2, chunk_pair, 0)
    for buf in range(2):
        pltpu.make_async_copy(
            out_v.at[buf], out_slab(N_CHUNK - 2 + buf), osems[buf]).wait()


def kernel(strands, strand_embed):
    s_t = strands.astype(jnp.int32).T
    tab = jnp.broadcast_to(
        strand_embed[:, :, None], (2, D_MODEL, LANES))
    out_t = _strand_encode(s_t, tab)
    return jnp.transpose(out_t, (2, 0, 1))
